# Initial kernel scaffold; baseline (speedup 1.0000x reference)
import jax
import jax.numpy as jnp
from jax.experimental import pallas as pl


def kernel(x, edge_index, W0, b0, W1, b1, W2, b2, W3, b3):
    raise NotImplementedError("write your pallas kernel here")


# speedup vs baseline: 7.3266x; 7.3266x over previous
import functools

import jax
import jax.numpy as jnp
from jax import lax
from jax.experimental import pallas as pl
from jax.experimental.pallas import tpu as pltpu
from jax.experimental.pallas import tpu_sc as plsc

N = 10000
D = 128
NP = 10240
NC = 2
NS = 16
NW = NC * NS
RPT = NP // NS
EB = 128
BLK = 1024
HW = 16


def _sc_mesh():
    return plsc.VectorSubcoreMesh(core_axis_name="c", subcore_axis_name="s")


@functools.lru_cache(maxsize=None)
def _hist_kernel(ch):
    @functools.partial(
        pl.kernel,
        out_type=jax.ShapeDtypeStruct((NC, NP, HW), jnp.float32),
        mesh=_sc_mesh(),
        scratch_types=[
            pltpu.VMEM((ch, EB), jnp.int32),
            pltpu.VMEM((EB, HW), jnp.float32),
            pltpu.VMEM_SHARED((NP, HW), jnp.float32),
        ],
    )
    def hist(dsti, zrows, ones, out, dstv, onesv, acc):
        c = lax.axis_index("c")
        s = lax.axis_index("s")
        w = c * NS + s
        sl = pl.ds(s * RPT, RPT)
        pltpu.sync_copy(zrows, acc.at[sl])
        pltpu.sync_copy(dsti.at[w], dstv)
        pltpu.sync_copy(ones, onesv)
        plsc.subcore_barrier()

        @pl.loop(0, ch)
        def _(j):
            pltpu.sync_copy(onesv, acc.at[dstv.at[j]], add=True)

        plsc.subcore_barrier()
        pltpu.sync_copy(acc.at[sl], out.at[c].at[sl])

    return hist


@functools.lru_cache(maxsize=None)
def _agg_kernel(ch):
    @functools.partial(
        pl.kernel,
        out_type=jax.ShapeDtypeStruct((NC, NP, D), jnp.float32),
        mesh=_sc_mesh(),
        scratch_types=[
            pltpu.VMEM((ch, EB), jnp.int32),
            pltpu.VMEM((ch, EB), jnp.int32),
            pltpu.VMEM((EB, D), jnp.float32),
            pltpu.VMEM_SHARED((NP, D), jnp.float32),
        ],
    )
    def agg(y, srci, dsti, zrows, out, srcv, dstv, rows, acc):
        c = lax.axis_index("c")
        s = lax.axis_index("s")
        w = c * NS + s
        sl = pl.ds(s * RPT, RPT)
        pltpu.sync_copy(zrows, acc.at[sl])
        pltpu.sync_copy(srci.at[w], srcv)
        pltpu.sync_copy(dsti.at[w], dstv)
        plsc.subcore_barrier()

        @pl.loop(0, ch)
        def _(j):
            pltpu.sync_copy(y.at[srcv.at[j]], rows)
            pltpu.sync_copy(rows, acc.at[dstv.at[j]], add=True)

        plsc.subcore_barrier()
        pltpu.sync_copy(acc.at[sl], out.at[c].at[sl])

    return agg


def _dinv_of(h_ref):
    deg = 1.0 + h_ref[0, :, :1] + h_ref[1, :, :1]
    return lax.rsqrt(deg)


def _mm_scale_body(x_ref, w_ref, h_ref, o_ref):
    dinv = _dinv_of(h_ref)
    o_ref[...] = jnp.dot(x_ref[...] * dinv, w_ref[...],
                         preferred_element_type=jnp.float32)


def _fuse_body(p_ref, y_ref, h_ref, b_ref, w_ref, o_ref):
    dinv = _dinv_of(h_ref)
    hcur = jnp.maximum((p_ref[0] + p_ref[1] + y_ref[...]) * dinv + b_ref[...],
                       0.0)
    o_ref[...] = jnp.dot(hcur * dinv, w_ref[...],
                         preferred_element_type=jnp.float32)


def _final_body(p_ref, y_ref, h_ref, b_ref, o_ref):
    dinv = _dinv_of(h_ref)
    o_ref[...] = jnp.maximum(
        (p_ref[0] + p_ref[1] + y_ref[...]) * dinv + b_ref[...], 0.0)


_ROW = pl.BlockSpec((BLK, D), lambda i: (i, 0))
_PAR = pl.BlockSpec((NC, BLK, D), lambda i: (0, i, 0))
_HIS = pl.BlockSpec((NC, BLK, HW), lambda i: (0, i, 0))
_WMAT = pl.BlockSpec((D, D), lambda i: (0, 0))
_BVEC = pl.BlockSpec((1, D), lambda i: (0, 0))
_OUT_SD = jax.ShapeDtypeStruct((NP, D), jnp.float32)
_GRID = (NP // BLK,)


def _mm_scale(x, w, hist):
    return pl.pallas_call(
        _mm_scale_body, grid=_GRID,
        in_specs=[_ROW, _WMAT, _HIS],
        out_specs=_ROW, out_shape=_OUT_SD,
    )(x, w, hist)


def _fuse(p, y, hist, b, w):
    return pl.pallas_call(
        _fuse_body, grid=_GRID,
        in_specs=[_PAR, _ROW, _HIS, _BVEC, _WMAT],
        out_specs=_ROW, out_shape=_OUT_SD,
    )(p, y, hist, b, w)


def _final(p, y, hist, b):
    return pl.pallas_call(
        _final_body, grid=_GRID,
        in_specs=[_PAR, _ROW, _HIS, _BVEC],
        out_specs=_ROW, out_shape=_OUT_SD,
    )(p, y, hist, b)


def kernel(x, edge_index, W0, b0, W1, b1, W2, b2, W3, b3):
    e = edge_index.shape[1]
    ch = -(-e // (NW * EB))
    ep = NW * ch * EB
    pad = jnp.full((ep - e,), N, dtype=jnp.int32)
    srci = jnp.concatenate([edge_index[0], pad]).reshape(NW, ch, EB)
    dsti = jnp.concatenate([edge_index[1], pad]).reshape(NW, ch, EB)

    x_pad = jnp.pad(x, ((0, NP - N), (0, 0)))
    z_agg = jnp.zeros((RPT, D), jnp.float32)
    z_his = jnp.zeros((RPT, HW), jnp.float32)
    ones = jnp.ones((EB, HW), jnp.float32)

    hist = _hist_kernel(ch)(dsti, z_his, ones)
    y = _mm_scale(x_pad, W0, hist)
    agg = _agg_kernel(ch)
    for b, w in ((b0, W1), (b1, W2), (b2, W3)):
        p = agg(y, srci, dsti, z_agg)
        y = _fuse(p, y, hist, b.reshape(1, D), w)
    p = agg(y, srci, dsti, z_agg)
    out = _final(p, y, hist, b3.reshape(1, D))
    return out[:N]
